# baseline (speedup 1.0000x reference)
import functools

import jax
import jax.numpy as jnp
from jax import lax
from jax.experimental import pallas as pl
from jax.experimental.pallas import tpu as pltpu
from jax.experimental.pallas import tpu_sc as plsc

_DIM_G = 16
_DIM_S = 32
_NC = 2
_NS = 16
_NW = _NC * _NS
_CHUNK = 128


@functools.lru_cache(maxsize=None)
def _build(B):
    b_per_w = B // _NW
    n_chunks = b_per_w // _CHUNK
    mesh = plsc.VectorSubcoreMesh(
        core_axis_name="c", subcore_axis_name="s",
        num_cores=_NC, num_subcores=_NS,
    )

    @functools.partial(
        pl.kernel,
        mesh=mesh,
        out_type=jax.ShapeDtypeStruct((B, _DIM_G + _DIM_S), jnp.float32),
        scratch_types=[
            pltpu.VMEM((n_chunks, _CHUNK), jnp.int32),
            pltpu.VMEM((n_chunks, _CHUNK), jnp.int32),
            pltpu.VMEM((b_per_w, _DIM_G), jnp.float32),
            pltpu.VMEM((b_per_w, _DIM_S), jnp.float32),
            pltpu.SemaphoreType.DMA,
        ],
        compiler_params=pltpu.CompilerParams(use_tc_tiling_on_sc=False),
    )
    def emb_kernel(gidx_hbm, sidx_hbm, wg_hbm, ws_hbm, out_hbm,
                   gidx_v, sidx_v, rows_g, rows_s, sem):
        wid = lax.axis_index("s") * _NC + lax.axis_index("c")
        base = wid * b_per_w
        crow = wid * n_chunks
        pltpu.sync_copy(gidx_hbm.at[pl.ds(crow, n_chunks)], gidx_v)
        pltpu.sync_copy(sidx_hbm.at[pl.ds(crow, n_chunks)], sidx_v)
        copies = []
        for j in range(n_chunks):
            copies.append(pltpu.async_copy(
                wg_hbm.at[gidx_v.at[j]],
                rows_g.at[pl.ds(j * _CHUNK, _CHUNK)], sem))
            copies.append(pltpu.async_copy(
                ws_hbm.at[sidx_v.at[j]],
                rows_s.at[pl.ds(j * _CHUNK, _CHUNK)], sem))
        for c in copies:
            c.wait()
        pltpu.sync_copy(rows_g, out_hbm.at[pl.ds(base, b_per_w), pl.ds(0, _DIM_G)])
        pltpu.sync_copy(rows_s, out_hbm.at[pl.ds(base, b_per_w), pl.ds(_DIM_G, _DIM_S)])

    return emb_kernel


@jax.jit
def kernel(graph_type, system_id, W_graph_type, W_system_id):
    B = graph_type.shape[0]
    gidx = graph_type.reshape(B // _CHUNK, _CHUNK)
    sidx = system_id.reshape(B // _CHUNK, _CHUNK)
    return _build(B)(gidx, sidx, W_graph_type, W_system_id)


# speedup vs baseline: 2.2060x; 2.2060x over previous
import functools

import jax
import jax.numpy as jnp
from jax import lax
from jax.experimental import pallas as pl
from jax.experimental.pallas import tpu as pltpu
from jax.experimental.pallas import tpu_sc as plsc

_DIM_G = 16
_DIM_S = 32
_DIM_O = _DIM_G + _DIM_S
_NC = 2
_NS = 16
_NW = _NC * _NS
_GRP = 16
_CROWS = 128


@functools.lru_cache(maxsize=None)
def _build(B):
    b_per_w = B // _NW
    n_grp = b_per_w // _GRP
    n_chunks = b_per_w // _CROWS
    grp_per_chunk = _CROWS // _GRP
    mesh = plsc.VectorSubcoreMesh(
        core_axis_name="c", subcore_axis_name="s",
        num_cores=_NC, num_subcores=_NS,
    )

    @functools.partial(
        pl.kernel,
        mesh=mesh,
        out_type=jax.ShapeDtypeStruct((B, _DIM_O), jnp.float32),
        scratch_types=[
            pltpu.VMEM((b_per_w,), jnp.int32),
            pltpu.VMEM((b_per_w,), jnp.int32),
            pltpu.VMEM((2, _GRP, 8, _DIM_G), jnp.float32),
            pltpu.VMEM((2, _GRP, 8, _DIM_S), jnp.float32),
            pltpu.VMEM((_CROWS, _DIM_O), jnp.float32),
            pltpu.SemaphoreType.DMA,
        ],
    )
    def emb_kernel(gidx_hbm, sidx_hbm, wg_hbm, ws_hbm, out_hbm,
                   gidx_v, sidx_v, g_buf, s_buf, out_c, sem):
        wid = lax.axis_index("s") * _NC + lax.axis_index("c")
        base = wid * b_per_w
        pltpu.sync_copy(gidx_hbm.at[pl.ds(base, b_per_w)], gidx_v)
        pltpu.sync_copy(sidx_hbm.at[pl.ds(base, b_per_w)], sidx_v)

        def fire(i, p):
            gvec = gidx_v[pl.ds(i * _GRP, _GRP)] >> 3
            svec = sidx_v[pl.ds(i * _GRP, _GRP)] >> 3
            for l in range(_GRP):
                pltpu.async_copy(wg_hbm.at[gvec[l]], g_buf.at[p, l], sem)
                pltpu.async_copy(ws_hbm.at[svec[l]], s_buf.at[p, l], sem)

        fire(0, 0)

        def step(i, _):
            p = lax.rem(i, 2)

            @pl.when(i + 1 < n_grp)
            def _():
                fire(i + 1, 1 - p)

            pltpu.make_async_copy(wg_hbm.at[pl.ds(0, _GRP)],
                                  g_buf.at[p], sem).wait()
            pltpu.make_async_copy(ws_hbm.at[pl.ds(0, _GRP)],
                                  s_buf.at[p], sem).wait()

            crow = lax.rem(i, grp_per_chunk) * _GRP
            gsub = lax.rem(gidx_v[pl.ds(i * _GRP, _GRP)], 8)
            ssub = lax.rem(sidx_v[pl.ds(i * _GRP, _GRP)], 8)
            for l in range(_GRP):
                row = crow + l
                out_c[row, pl.ds(0, _DIM_G)] = g_buf[p, l, gsub[l], :]
                out_c[row, pl.ds(_DIM_G, 16)] = s_buf[p, l, ssub[l], pl.ds(0, 16)]
                out_c[row, pl.ds(_DIM_G + 16, 16)] = s_buf[p, l, ssub[l], pl.ds(16, 16)]
            return ()

        for c in range(n_chunks):
            lax.fori_loop(c * grp_per_chunk, (c + 1) * grp_per_chunk,
                          step, (), unroll=False)
            pltpu.sync_copy(out_c, out_hbm.at[pl.ds(base + c * _CROWS, _CROWS)])

    return emb_kernel


@jax.jit
def kernel(graph_type, system_id, W_graph_type, W_system_id):
    B = graph_type.shape[0]
    ng, dg = W_graph_type.shape
    ns, ds = W_system_id.shape
    return _build(B)(
        graph_type.reshape(B), system_id.reshape(B),
        W_graph_type.reshape(ng // 8, 8, dg),
        W_system_id.reshape(ns // 8, 8, ds),
    )
